# baseline (speedup 1.0000x reference)
import jax
import jax.numpy as jnp
from jax import lax
from jax.experimental import pallas as pl
from jax.experimental.pallas import tpu as pltpu
from jax.experimental.pallas import tpu_sc as plsc

N = 10000
E = 320000
H = 128
C = 16
G = 16

NC = 2
NS = 16
NW = NC * NS

CHUNK = 128
NCHUNKS = E // CHUNK
FULL_TRIPS = NCHUNKS // NW
EXTRA = NCHUNKS - FULL_TRIPS * NW
N_PAD = 10112
ROWS_PER_TILE = N_PAD // NS
MAX_EDGES_W = (FULL_TRIPS + 1) * CHUNK
SPANS = ((0, 128), (128, 128), (256, 128), (384, 128), (512, 120))
CNT_PAD = 10240
CNT_PER_TILE = CNT_PAD // NS

_MESH = plsc.VectorSubcoreMesh(core_axis_name="c", subcore_axis_name="s")


def _fill_1d(ref, nwords, value):
    v = jnp.full((16,), value, jnp.float32)

    def body(j, carry):
        ref[pl.ds(j * 16, 16)] = v
        return carry

    lax.fori_loop(0, nwords // 16, body, 0)


def _zero_fill_2d(ref, nrows):
    zv = jnp.zeros((16,), jnp.float32)

    def body(t, carry):
        r = t // (H // 16)
        c = (t % (H // 16)) * 16
        ref[r, pl.ds(c, 16)] = zv
        return carry

    lax.fori_loop(0, nrows * (H // 16), body, 0)


def _zero_spmem_slice(rows0, sp_agg, sid):
    _zero_fill_2d(rows0, 128)
    for off, nr in SPANS:
        pltpu.sync_copy(rows0.at[pl.ds(0, nr)],
                        sp_agg.at[pl.ds(sid * ROWS_PER_TILE + off, nr)])


def _copy_out(sp_agg, agg_out, cid, sid, sem):
    for off, nr in SPANS:
        o = sid * ROWS_PER_TILE + off
        pltpu.async_copy(sp_agg.at[pl.ds(o, nr)],
                         agg_out.at[cid, pl.ds(o, nr)], sem)
    for off, nr in SPANS:
        o = sid * ROWS_PER_TILE + off
        pltpu.make_async_copy(sp_agg.at[pl.ds(o, nr)],
                              agg_out.at[cid, pl.ds(o, nr)], sem).wait()


def _sc_agg_cnt_body(h_hbm, src_hbm, dst_hbm, agg_out, cnt_out,
                     src_all, dst2, rows, ones_v,
                     sp_agg, sp_cnt, isem, gsem, ssem):
    cid = lax.axis_index("c")
    sid = lax.axis_index("s")
    wid = sid * NC + cid

    trips = jnp.where(wid < EXTRA, FULL_TRIPS + 1, FULL_TRIPS)
    base = (FULL_TRIPS * wid + jnp.minimum(wid, EXTRA)) * CHUNK

    nmain = FULL_TRIPS * CHUNK
    i0 = pltpu.async_copy(src_hbm.at[pl.ds(base, nmain)],
                          src_all.at[pl.ds(0, nmain)], isem)

    @pl.when(wid < EXTRA)
    def _():
        pltpu.async_copy(src_hbm.at[pl.ds(base + nmain, CHUNK)],
                         src_all.at[pl.ds(nmain, CHUNK)], isem).wait()

    _zero_spmem_slice(rows.at[0], sp_agg, sid)
    _fill_1d(ones_v, CHUNK, 1.0)
    for k in range(5):
        pltpu.sync_copy(rows.at[0, 0],
                        sp_cnt.at[pl.ds(sid * CNT_PER_TILE + k * 128, 128)])
    i0.wait()

    plsc.subcore_barrier()

    def start_chunk(t, b):
        pltpu.async_copy(dst_hbm.at[pl.ds(base + t * CHUNK, CHUNK)],
                         dst2.at[b], isem)
        pltpu.async_copy(h_hbm.at[src_all.at[pl.ds(t * CHUNK, CHUNK)]],
                         rows.at[b], gsem)

    def wait_chunk(t, b):
        pltpu.make_async_copy(dst_hbm.at[pl.ds(base + t * CHUNK, CHUNK)],
                              dst2.at[b], isem).wait()
        pltpu.make_async_copy(h_hbm.at[src_all.at[pl.ds(t * CHUNK, CHUNK)]],
                              rows.at[b], gsem).wait()

    def wait_scatter(b):
        pltpu.make_async_copy(rows.at[b], sp_agg.at[dst2.at[b]],
                              ssem).wait()

    start_chunk(0, 0)

    def body(t, carry):
        b = t % 2

        @pl.when(t >= 1)
        def _():
            wait_scatter(1 - b)

        @pl.when(t + 1 < trips)
        def _():
            start_chunk(t + 1, 1 - b)

        wait_chunk(t, b)
        pltpu.async_copy(rows.at[b], sp_agg.at[dst2.at[b]], ssem, add=True)
        pltpu.sync_copy(ones_v, sp_cnt.at[dst2.at[b]], add=True)
        return carry

    lax.fori_loop(0, trips, body, 0)
    wait_scatter((trips - 1) % 2)

    plsc.subcore_barrier()

    _copy_out(sp_agg, agg_out, cid, sid, gsem)
    pltpu.sync_copy(sp_cnt.at[pl.ds(sid * CNT_PER_TILE, CNT_PER_TILE)],
                    cnt_out.at[pl.ds(cid * CNT_PAD + sid * CNT_PER_TILE,
                                     CNT_PER_TILE)])


def _sc_agg_body(h_hbm, src_hbm, dst_hbm, agg_out,
                 src4, dst4, rows, sp_agg, isem, gsem, ssem):
    cid = lax.axis_index("c")
    sid = lax.axis_index("s")
    wid = sid * NC + cid

    trips = jnp.where(wid < EXTRA, FULL_TRIPS + 1, FULL_TRIPS)
    base = (FULL_TRIPS * wid + jnp.minimum(wid, EXTRA)) * CHUNK

    def start_idx(t):
        j = t % 4
        pltpu.async_copy(src_hbm.at[pl.ds(base + t * CHUNK, CHUNK)],
                         src4.at[j], isem)
        pltpu.async_copy(dst_hbm.at[pl.ds(base + t * CHUNK, CHUNK)],
                         dst4.at[j], isem)

    def wait_idx(t):
        j = t % 4
        pltpu.make_async_copy(src_hbm.at[pl.ds(base + t * CHUNK, CHUNK)],
                              src4.at[j], isem).wait()
        pltpu.make_async_copy(dst_hbm.at[pl.ds(base + t * CHUNK, CHUNK)],
                              dst4.at[j], isem).wait()

    def start_gather(t):
        pltpu.async_copy(h_hbm.at[src4.at[t % 4]], rows.at[t % 3], gsem)

    def wait_gather(t):
        pltpu.make_async_copy(h_hbm.at[src4.at[t % 4]], rows.at[t % 3],
                              gsem).wait()

    def start_scatter(t):
        pltpu.async_copy(rows.at[t % 3], sp_agg.at[dst4.at[t % 4]], ssem,
                         add=True)

    def wait_scatter(t):
        pltpu.make_async_copy(rows.at[t % 3], sp_agg.at[dst4.at[t % 4]],
                              ssem).wait()

    start_idx(0)
    start_idx(1)
    _zero_spmem_slice(rows.at[0], sp_agg, sid)
    wait_idx(0)
    start_gather(0)

    plsc.subcore_barrier()

    def body(t, carry):
        @pl.when(t >= 2)
        def _():
            wait_scatter(t - 2)

        @pl.when(t + 2 < trips)
        def _():
            start_idx(t + 2)

        @pl.when(t + 1 < trips)
        def _():
            wait_idx(t + 1)
            start_gather(t + 1)

        wait_gather(t)
        start_scatter(t)
        return carry

    lax.fori_loop(0, trips, body, 0)

    @pl.when(trips >= 2)
    def _():
        wait_scatter(trips - 2)

    wait_scatter(trips - 1)

    plsc.subcore_barrier()

    _copy_out(sp_agg, agg_out, cid, sid, gsem)


_sc_agg_cnt = pl.kernel(
    _sc_agg_cnt_body,
    out_type=(jax.ShapeDtypeStruct((NC, N_PAD, H), jnp.float32),
              jax.ShapeDtypeStruct((NC * CNT_PAD,), jnp.float32)),
    mesh=_MESH,
    scratch_types=[
        pltpu.VMEM((MAX_EDGES_W,), jnp.int32),
        pltpu.VMEM((2, CHUNK), jnp.int32),
        pltpu.VMEM((2, CHUNK, H), jnp.float32),
        pltpu.VMEM((CHUNK,), jnp.float32),
        pltpu.VMEM_SHARED((N_PAD, H), jnp.float32),
        pltpu.VMEM_SHARED((CNT_PAD,), jnp.float32),
        pltpu.SemaphoreType.DMA,
        pltpu.SemaphoreType.DMA,
        pltpu.SemaphoreType.DMA,
    ],
)

_sc_agg = pl.kernel(
    _sc_agg_body,
    out_type=jax.ShapeDtypeStruct((NC, N_PAD, H), jnp.float32),
    mesh=_MESH,
    scratch_types=[
        pltpu.VMEM((4, CHUNK), jnp.int32),
        pltpu.VMEM((4, CHUNK), jnp.int32),
        pltpu.VMEM((3, CHUNK, H), jnp.float32),
        pltpu.VMEM_SHARED((N_PAD, H), jnp.float32),
        pltpu.SemaphoreType.DMA,
        pltpu.SemaphoreType.DMA,
        pltpu.SemaphoreType.DMA,
    ],
)


def _layer_body(aggp, h, inv, wl, bl, wr, g, b, out):
    mean = (aggp[0][:N] + aggp[1][:N]) * inv[...]
    z = (jnp.dot(mean, wl[...], preferred_element_type=jnp.float32)
         + jnp.dot(h[...], wr[...], preferred_element_type=jnp.float32)
         + bl[...])
    r = jnp.maximum(z, 0.0)
    m = jnp.mean(r, axis=0, keepdims=True)
    d = r - m
    v = jnp.mean(d * d, axis=0, keepdims=True)
    out[...] = g[...] * d / jnp.sqrt(v + 1e-5) + b[...]


def _final_body(h, batch, ones_n, w1, b1, w2, b2, out):
    onehot = (batch[...] == lax.broadcasted_iota(jnp.int32, (N, G), 1)
              ).astype(jnp.float32)
    dnums = (((0,), (0,)), ((), ()))
    psum = lax.dot_general(onehot, h[...], dnums,
                           preferred_element_type=jnp.float32)
    gcnt = lax.dot_general(onehot, ones_n[...], dnums,
                           preferred_element_type=jnp.float32)
    pooled = psum / jnp.maximum(gcnt, 1.0)
    h2 = jnp.maximum(
        jnp.dot(pooled, w1[...], preferred_element_type=jnp.float32)
        + b1[...], 0.0)
    logits = (jnp.dot(h2, w2[...], preferred_element_type=jnp.float32)
              + b2[...])
    mx = jnp.max(logits, axis=-1, keepdims=True)
    s = logits - mx
    lse = jnp.log(jnp.sum(jnp.exp(s), axis=-1, keepdims=True))
    out[...] = s - lse


_tc_layer = pl.pallas_call(
    _layer_body,
    out_shape=jax.ShapeDtypeStruct((N, H), jnp.float32),
)

_tc_final = pl.pallas_call(
    _final_body,
    out_shape=jax.ShapeDtypeStruct((G, C), jnp.float32),
)


def kernel(x, edge_index, batch, params):
    src = edge_index[0]
    dst = edge_index[1]
    ones_n = jnp.ones((N, 1), jnp.float32)
    batch2d = batch.reshape(N, 1)

    def w(i):
        p = params['conv%d' % i]
        return (p['Wl'], p['bl'].reshape(1, H), p['Wr'],
                params['bn%d_g' % i].reshape(1, H),
                params['bn%d_b' % i].reshape(1, H))

    aggp, cntp = _sc_agg_cnt(x, src, dst)
    inv = (1.0 / jnp.maximum(cntp[:N] + cntp[CNT_PAD:CNT_PAD + N],
                             1.0)).reshape(N, 1)
    h = x
    for i in range(1, 8):
        if i > 1:
            aggp = _sc_agg(h, src, dst)
        wl, bl, wr, g, b = w(i)
        h = _tc_layer(aggp, h, inv, wl, bl, wr, g, b)

    return _tc_final(h, batch2d, ones_n,
                     params['fc1_W'], params['fc1_b'].reshape(1, H),
                     params['fc2_W'], params['fc2_b'].reshape(1, C))


# speedup vs baseline: 1.0084x; 1.0084x over previous
import jax
import jax.numpy as jnp
from jax import lax
from jax.experimental import pallas as pl
from jax.experimental.pallas import tpu as pltpu
from jax.experimental.pallas import tpu_sc as plsc

N = 10000
E = 320000
H = 128
C = 16
G = 16

NC = 2
NS = 16
NW = NC * NS

CHUNK = 128
NCHUNKS = E // CHUNK
FULL_TRIPS = NCHUNKS // NW
EXTRA = NCHUNKS - FULL_TRIPS * NW
N_PAD = 10112
ROWS_PER_TILE = N_PAD // NS
MAX_EDGES_W = (FULL_TRIPS + 1) * CHUNK
SPANS = ((0, 128), (128, 128), (256, 128), (384, 128), (512, 120))
CNT_PAD = 10240
CNT_PER_TILE = CNT_PAD // NS

_MESH = plsc.VectorSubcoreMesh(core_axis_name="c", subcore_axis_name="s")


def _fill_1d(ref, nwords, value):
    v = jnp.full((16,), value, jnp.float32)

    def body(j, carry):
        ref[pl.ds(j * 16, 16)] = v
        return carry

    lax.fori_loop(0, nwords // 16, body, 0)


def _zero_fill_2d(ref, nrows):
    zv = jnp.zeros((16,), jnp.float32)

    def body(t, carry):
        r = t // (H // 16)
        c = (t % (H // 16)) * 16
        ref[r, pl.ds(c, 16)] = zv
        return carry

    lax.fori_loop(0, nrows * (H // 16), body, 0)


def _zero_spmem_slice(rows0, sp_agg, sid):
    _zero_fill_2d(rows0, 128)
    for off, nr in SPANS:
        pltpu.sync_copy(rows0.at[pl.ds(0, nr)],
                        sp_agg.at[pl.ds(sid * ROWS_PER_TILE + off, nr)])


def _copy_out(sp_agg, agg_out, cid, sid, sem):
    for off, nr in SPANS:
        o = sid * ROWS_PER_TILE + off
        pltpu.async_copy(sp_agg.at[pl.ds(o, nr)],
                         agg_out.at[cid, pl.ds(o, nr)], sem)
    for off, nr in SPANS:
        o = sid * ROWS_PER_TILE + off
        pltpu.make_async_copy(sp_agg.at[pl.ds(o, nr)],
                              agg_out.at[cid, pl.ds(o, nr)], sem).wait()


def _sc_agg_cnt_body(h_hbm, src_hbm, dst_hbm, agg_out, cnt_out,
                     src_all, dst2, rows, ones_v,
                     sp_agg, sp_cnt, isem, gsem, ssem, csem):
    cid = lax.axis_index("c")
    sid = lax.axis_index("s")
    wid = sid * NC + cid

    trips = jnp.where(wid < EXTRA, FULL_TRIPS + 1, FULL_TRIPS)
    base = (FULL_TRIPS * wid + jnp.minimum(wid, EXTRA)) * CHUNK

    nmain = FULL_TRIPS * CHUNK
    i0 = pltpu.async_copy(src_hbm.at[pl.ds(base, nmain)],
                          src_all.at[pl.ds(0, nmain)], isem)

    @pl.when(wid < EXTRA)
    def _():
        pltpu.async_copy(src_hbm.at[pl.ds(base + nmain, CHUNK)],
                         src_all.at[pl.ds(nmain, CHUNK)], isem).wait()

    _zero_spmem_slice(rows.at[0], sp_agg, sid)
    _fill_1d(ones_v, CHUNK, 1.0)
    for k in range(5):
        pltpu.sync_copy(rows.at[0, 0],
                        sp_cnt.at[pl.ds(sid * CNT_PER_TILE + k * 128, 128)])
    i0.wait()

    plsc.subcore_barrier()

    def start_chunk(t, b):
        pltpu.async_copy(dst_hbm.at[pl.ds(base + t * CHUNK, CHUNK)],
                         dst2.at[b], isem)
        pltpu.async_copy(h_hbm.at[src_all.at[pl.ds(t * CHUNK, CHUNK)]],
                         rows.at[b], gsem)

    def wait_chunk(t, b):
        pltpu.make_async_copy(dst_hbm.at[pl.ds(base + t * CHUNK, CHUNK)],
                              dst2.at[b], isem).wait()
        pltpu.make_async_copy(h_hbm.at[src_all.at[pl.ds(t * CHUNK, CHUNK)]],
                              rows.at[b], gsem).wait()

    def wait_scatter(b):
        pltpu.make_async_copy(rows.at[b], sp_agg.at[dst2.at[b]],
                              ssem).wait()
        pltpu.make_async_copy(ones_v, sp_cnt.at[dst2.at[b]], csem).wait()

    start_chunk(0, 0)

    def body(t, carry):
        b = t % 2

        @pl.when(t >= 1)
        def _():
            wait_scatter(1 - b)

        @pl.when(t + 1 < trips)
        def _():
            start_chunk(t + 1, 1 - b)

        wait_chunk(t, b)
        pltpu.async_copy(rows.at[b], sp_agg.at[dst2.at[b]], ssem, add=True)
        pltpu.async_copy(ones_v, sp_cnt.at[dst2.at[b]], csem, add=True)
        return carry

    lax.fori_loop(0, trips, body, 0)
    wait_scatter((trips - 1) % 2)

    plsc.subcore_barrier()

    _copy_out(sp_agg, agg_out, cid, sid, gsem)
    pltpu.sync_copy(sp_cnt.at[pl.ds(sid * CNT_PER_TILE, CNT_PER_TILE)],
                    cnt_out.at[pl.ds(cid * CNT_PAD + sid * CNT_PER_TILE,
                                     CNT_PER_TILE)])


def _sc_agg_body(h_hbm, src_hbm, dst_hbm, agg_out,
                 src4, dst4, rows, sp_agg, isem, gsem, ssem):
    cid = lax.axis_index("c")
    sid = lax.axis_index("s")
    wid = sid * NC + cid

    trips = jnp.where(wid < EXTRA, FULL_TRIPS + 1, FULL_TRIPS)
    base = (FULL_TRIPS * wid + jnp.minimum(wid, EXTRA)) * CHUNK

    def start_idx(t):
        j = t % 4
        pltpu.async_copy(src_hbm.at[pl.ds(base + t * CHUNK, CHUNK)],
                         src4.at[j], isem)
        pltpu.async_copy(dst_hbm.at[pl.ds(base + t * CHUNK, CHUNK)],
                         dst4.at[j], isem)

    def wait_idx(t):
        j = t % 4
        pltpu.make_async_copy(src_hbm.at[pl.ds(base + t * CHUNK, CHUNK)],
                              src4.at[j], isem).wait()
        pltpu.make_async_copy(dst_hbm.at[pl.ds(base + t * CHUNK, CHUNK)],
                              dst4.at[j], isem).wait()

    def start_gather(t):
        pltpu.async_copy(h_hbm.at[src4.at[t % 4]], rows.at[t % 3], gsem)

    def wait_gather(t):
        pltpu.make_async_copy(h_hbm.at[src4.at[t % 4]], rows.at[t % 3],
                              gsem).wait()

    def start_scatter(t):
        pltpu.async_copy(rows.at[t % 3], sp_agg.at[dst4.at[t % 4]], ssem,
                         add=True)

    def wait_scatter(t):
        pltpu.make_async_copy(rows.at[t % 3], sp_agg.at[dst4.at[t % 4]],
                              ssem).wait()

    start_idx(0)
    start_idx(1)
    _zero_spmem_slice(rows.at[0], sp_agg, sid)
    wait_idx(0)
    start_gather(0)

    plsc.subcore_barrier()

    def body(t, carry):
        @pl.when(t >= 2)
        def _():
            wait_scatter(t - 2)

        @pl.when(t + 2 < trips)
        def _():
            start_idx(t + 2)

        @pl.when(t + 1 < trips)
        def _():
            wait_idx(t + 1)
            start_gather(t + 1)

        wait_gather(t)
        start_scatter(t)
        return carry

    lax.fori_loop(0, trips, body, 0)

    @pl.when(trips >= 2)
    def _():
        wait_scatter(trips - 2)

    wait_scatter(trips - 1)

    plsc.subcore_barrier()

    _copy_out(sp_agg, agg_out, cid, sid, gsem)


_sc_agg_cnt = pl.kernel(
    _sc_agg_cnt_body,
    out_type=(jax.ShapeDtypeStruct((NC, N_PAD, H), jnp.float32),
              jax.ShapeDtypeStruct((NC * CNT_PAD,), jnp.float32)),
    mesh=_MESH,
    scratch_types=[
        pltpu.VMEM((MAX_EDGES_W,), jnp.int32),
        pltpu.VMEM((2, CHUNK), jnp.int32),
        pltpu.VMEM((2, CHUNK, H), jnp.float32),
        pltpu.VMEM((CHUNK,), jnp.float32),
        pltpu.VMEM_SHARED((N_PAD, H), jnp.float32),
        pltpu.VMEM_SHARED((CNT_PAD,), jnp.float32),
        pltpu.SemaphoreType.DMA,
        pltpu.SemaphoreType.DMA,
        pltpu.SemaphoreType.DMA,
        pltpu.SemaphoreType.DMA,
    ],
)

_sc_agg = pl.kernel(
    _sc_agg_body,
    out_type=jax.ShapeDtypeStruct((NC, N_PAD, H), jnp.float32),
    mesh=_MESH,
    scratch_types=[
        pltpu.VMEM((4, CHUNK), jnp.int32),
        pltpu.VMEM((4, CHUNK), jnp.int32),
        pltpu.VMEM((3, CHUNK, H), jnp.float32),
        pltpu.VMEM_SHARED((N_PAD, H), jnp.float32),
        pltpu.SemaphoreType.DMA,
        pltpu.SemaphoreType.DMA,
        pltpu.SemaphoreType.DMA,
    ],
)


def _layer_body(aggp, h, inv, wl, bl, wr, g, b, out):
    mean = (aggp[0][:N] + aggp[1][:N]) * inv[...]
    z = (jnp.dot(mean, wl[...], preferred_element_type=jnp.float32)
         + jnp.dot(h[...], wr[...], preferred_element_type=jnp.float32)
         + bl[...])
    r = jnp.maximum(z, 0.0)
    m = jnp.mean(r, axis=0, keepdims=True)
    d = r - m
    v = jnp.mean(d * d, axis=0, keepdims=True)
    out[...] = g[...] * d / jnp.sqrt(v + 1e-5) + b[...]


def _layer7_final_body(aggp, h, inv, wl, bl, wr, g, b,
                       batch, ones_n, w1, b1, w2, b2, out):
    mean = (aggp[0][:N] + aggp[1][:N]) * inv[...]
    z = (jnp.dot(mean, wl[...], preferred_element_type=jnp.float32)
         + jnp.dot(h[...], wr[...], preferred_element_type=jnp.float32)
         + bl[...])
    r = jnp.maximum(z, 0.0)
    m = jnp.mean(r, axis=0, keepdims=True)
    d = r - m
    v = jnp.mean(d * d, axis=0, keepdims=True)
    h7 = g[...] * d / jnp.sqrt(v + 1e-5) + b[...]
    onehot = (batch[...] == lax.broadcasted_iota(jnp.int32, (N, G), 1)
              ).astype(jnp.float32)
    dnums = (((0,), (0,)), ((), ()))
    psum = lax.dot_general(onehot, h7, dnums,
                           preferred_element_type=jnp.float32)
    gcnt = lax.dot_general(onehot, ones_n[...], dnums,
                           preferred_element_type=jnp.float32)
    pooled = psum / jnp.maximum(gcnt, 1.0)
    h2 = jnp.maximum(
        jnp.dot(pooled, w1[...], preferred_element_type=jnp.float32)
        + b1[...], 0.0)
    logits = (jnp.dot(h2, w2[...], preferred_element_type=jnp.float32)
              + b2[...])
    mx = jnp.max(logits, axis=-1, keepdims=True)
    s = logits - mx
    lse = jnp.log(jnp.sum(jnp.exp(s), axis=-1, keepdims=True))
    out[...] = s - lse


_tc_layer = pl.pallas_call(
    _layer_body,
    out_shape=jax.ShapeDtypeStruct((N, H), jnp.float32),
)

_tc_layer7_final = pl.pallas_call(
    _layer7_final_body,
    out_shape=jax.ShapeDtypeStruct((G, C), jnp.float32),
)


def kernel(x, edge_index, batch, params):
    src = edge_index[0]
    dst = edge_index[1]
    ones_n = jnp.ones((N, 1), jnp.float32)
    batch2d = batch.reshape(N, 1)

    def w(i):
        p = params['conv%d' % i]
        return (p['Wl'], p['bl'].reshape(1, H), p['Wr'],
                params['bn%d_g' % i].reshape(1, H),
                params['bn%d_b' % i].reshape(1, H))

    aggp, cntp = _sc_agg_cnt(x, src, dst)
    inv = (1.0 / jnp.maximum(cntp[:N] + cntp[CNT_PAD:CNT_PAD + N],
                             1.0)).reshape(N, 1)
    h = x
    for i in range(1, 7):
        if i > 1:
            aggp = _sc_agg(h, src, dst)
        wl, bl, wr, g, b = w(i)
        h = _tc_layer(aggp, h, inv, wl, bl, wr, g, b)

    aggp = _sc_agg(h, src, dst)
    wl, bl, wr, g, b = w(7)
    return _tc_layer7_final(aggp, h, inv, wl, bl, wr, g, b,
                            batch2d, ones_n,
                            params['fc1_W'], params['fc1_b'].reshape(1, H),
                            params['fc2_W'], params['fc2_b'].reshape(1, C))


# speedup vs baseline: 1.0387x; 1.0301x over previous
import jax
import jax.numpy as jnp
from jax import lax
from jax.experimental import pallas as pl
from jax.experimental.pallas import tpu as pltpu
from jax.experimental.pallas import tpu_sc as plsc

N = 10000
E = 320000
H = 128
C = 16
G = 16

NC = 2
NS = 16
NW = NC * NS

CHUNK = 128
NCHUNKS = E // CHUNK
FULL_TRIPS = NCHUNKS // NW
EXTRA = NCHUNKS - FULL_TRIPS * NW
N_PAD = 10112
ROWS_PER_TILE = N_PAD // NS
MAX_EDGES_W = (FULL_TRIPS + 1) * CHUNK
SPANS = ((0, 128), (128, 128), (256, 128), (384, 128), (512, 120))
CNT_PAD = 10240
CNT_PER_TILE = CNT_PAD // NS

_MESH = plsc.VectorSubcoreMesh(core_axis_name="c", subcore_axis_name="s")


def _fill_1d(ref, nwords, value):
    v = jnp.full((16,), value, jnp.float32)

    def body(j, carry):
        ref[pl.ds(j * 16, 16)] = v
        return carry

    lax.fori_loop(0, nwords // 16, body, 0)


def _zero_fill_2d(ref, nrows):
    zv = jnp.zeros((16,), jnp.float32)

    def body(r, carry):
        for j in range(H // 16):
            ref[r, pl.ds(j * 16, 16)] = zv
        return carry

    lax.fori_loop(0, nrows, body, 0)


def _zero_spmem_slice(rows0, sp_agg, sid, sem):
    _zero_fill_2d(rows0, 128)
    for off, nr in SPANS:
        pltpu.async_copy(rows0.at[pl.ds(0, nr)],
                         sp_agg.at[pl.ds(sid * ROWS_PER_TILE + off, nr)],
                         sem)
    for off, nr in SPANS:
        pltpu.make_async_copy(
            rows0.at[pl.ds(0, nr)],
            sp_agg.at[pl.ds(sid * ROWS_PER_TILE + off, nr)], sem).wait()


def _copy_out(sp_agg, agg_out, cid, sid, sem):
    for off, nr in SPANS:
        o = sid * ROWS_PER_TILE + off
        pltpu.async_copy(sp_agg.at[pl.ds(o, nr)],
                         agg_out.at[cid, pl.ds(o, nr)], sem)
    for off, nr in SPANS:
        o = sid * ROWS_PER_TILE + off
        pltpu.make_async_copy(sp_agg.at[pl.ds(o, nr)],
                              agg_out.at[cid, pl.ds(o, nr)], sem).wait()


def _sc_agg_cnt_body(h_hbm, src_hbm, dst_hbm, agg_out, cnt_out,
                     src_all, dst2, rows, ones_v,
                     sp_agg, sp_cnt, isem, gsem, ssem, csem):
    cid = lax.axis_index("c")
    sid = lax.axis_index("s")
    wid = sid * NC + cid

    trips = jnp.where(wid < EXTRA, FULL_TRIPS + 1, FULL_TRIPS)
    base = (FULL_TRIPS * wid + jnp.minimum(wid, EXTRA)) * CHUNK

    nmain = FULL_TRIPS * CHUNK
    i0 = pltpu.async_copy(src_hbm.at[pl.ds(base, nmain)],
                          src_all.at[pl.ds(0, nmain)], isem)

    @pl.when(wid < EXTRA)
    def _():
        pltpu.async_copy(src_hbm.at[pl.ds(base + nmain, CHUNK)],
                         src_all.at[pl.ds(nmain, CHUNK)], isem).wait()

    _zero_spmem_slice(rows.at[0], sp_agg, sid, gsem)
    _fill_1d(ones_v, CHUNK, 1.0)
    for k in range(5):
        pltpu.sync_copy(rows.at[0, 0],
                        sp_cnt.at[pl.ds(sid * CNT_PER_TILE + k * 128, 128)])
    i0.wait()

    plsc.subcore_barrier()

    def start_chunk(t, b):
        pltpu.async_copy(dst_hbm.at[pl.ds(base + t * CHUNK, CHUNK)],
                         dst2.at[b], isem)
        pltpu.async_copy(h_hbm.at[src_all.at[pl.ds(t * CHUNK, CHUNK)]],
                         rows.at[b], gsem)

    def wait_chunk(t, b):
        pltpu.make_async_copy(dst_hbm.at[pl.ds(base + t * CHUNK, CHUNK)],
                              dst2.at[b], isem).wait()
        pltpu.make_async_copy(h_hbm.at[src_all.at[pl.ds(t * CHUNK, CHUNK)]],
                              rows.at[b], gsem).wait()

    def wait_scatter(b):
        pltpu.make_async_copy(rows.at[b], sp_agg.at[dst2.at[b]],
                              ssem).wait()
        pltpu.make_async_copy(ones_v, sp_cnt.at[dst2.at[b]], csem).wait()

    start_chunk(0, 0)

    def body(t, carry):
        b = t % 2

        @pl.when(t >= 1)
        def _():
            wait_scatter(1 - b)

        @pl.when(t + 1 < trips)
        def _():
            start_chunk(t + 1, 1 - b)

        wait_chunk(t, b)
        pltpu.async_copy(rows.at[b], sp_agg.at[dst2.at[b]], ssem, add=True)
        pltpu.async_copy(ones_v, sp_cnt.at[dst2.at[b]], csem, add=True)
        return carry

    lax.fori_loop(0, trips, body, 0)
    wait_scatter((trips - 1) % 2)

    plsc.subcore_barrier()

    _copy_out(sp_agg, agg_out, cid, sid, gsem)
    pltpu.sync_copy(sp_cnt.at[pl.ds(sid * CNT_PER_TILE, CNT_PER_TILE)],
                    cnt_out.at[pl.ds(cid * CNT_PAD + sid * CNT_PER_TILE,
                                     CNT_PER_TILE)])


def _sc_agg_body(h_hbm, src_hbm, dst_hbm, agg_out,
                 src4, dst4, rows, sp_agg, isem, gsem, ssem):
    cid = lax.axis_index("c")
    sid = lax.axis_index("s")
    wid = sid * NC + cid

    trips = jnp.where(wid < EXTRA, FULL_TRIPS + 1, FULL_TRIPS)
    base = (FULL_TRIPS * wid + jnp.minimum(wid, EXTRA)) * CHUNK

    def start_idx(t):
        j = t % 4
        pltpu.async_copy(src_hbm.at[pl.ds(base + t * CHUNK, CHUNK)],
                         src4.at[j], isem)
        pltpu.async_copy(dst_hbm.at[pl.ds(base + t * CHUNK, CHUNK)],
                         dst4.at[j], isem)

    def wait_idx(t):
        j = t % 4
        pltpu.make_async_copy(src_hbm.at[pl.ds(base + t * CHUNK, CHUNK)],
                              src4.at[j], isem).wait()
        pltpu.make_async_copy(dst_hbm.at[pl.ds(base + t * CHUNK, CHUNK)],
                              dst4.at[j], isem).wait()

    def start_gather(t):
        pltpu.async_copy(h_hbm.at[src4.at[t % 4]], rows.at[t % 3], gsem)

    def wait_gather(t):
        pltpu.make_async_copy(h_hbm.at[src4.at[t % 4]], rows.at[t % 3],
                              gsem).wait()

    def start_scatter(t):
        pltpu.async_copy(rows.at[t % 3], sp_agg.at[dst4.at[t % 4]], ssem,
                         add=True)

    def wait_scatter(t):
        pltpu.make_async_copy(rows.at[t % 3], sp_agg.at[dst4.at[t % 4]],
                              ssem).wait()

    start_idx(0)
    start_idx(1)
    _zero_spmem_slice(rows.at[0], sp_agg, sid, gsem)
    wait_idx(0)
    start_gather(0)

    plsc.subcore_barrier()

    def body(t, carry):
        @pl.when(t >= 2)
        def _():
            wait_scatter(t - 2)

        @pl.when(t + 2 < trips)
        def _():
            start_idx(t + 2)

        @pl.when(t + 1 < trips)
        def _():
            wait_idx(t + 1)
            start_gather(t + 1)

        wait_gather(t)
        start_scatter(t)
        return carry

    lax.fori_loop(0, trips, body, 0)

    @pl.when(trips >= 2)
    def _():
        wait_scatter(trips - 2)

    wait_scatter(trips - 1)

    plsc.subcore_barrier()

    _copy_out(sp_agg, agg_out, cid, sid, gsem)


_sc_agg_cnt = pl.kernel(
    _sc_agg_cnt_body,
    out_type=(jax.ShapeDtypeStruct((NC, N_PAD, H), jnp.float32),
              jax.ShapeDtypeStruct((NC * CNT_PAD,), jnp.float32)),
    mesh=_MESH,
    scratch_types=[
        pltpu.VMEM((MAX_EDGES_W,), jnp.int32),
        pltpu.VMEM((2, CHUNK), jnp.int32),
        pltpu.VMEM((2, CHUNK, H), jnp.float32),
        pltpu.VMEM((CHUNK,), jnp.float32),
        pltpu.VMEM_SHARED((N_PAD, H), jnp.float32),
        pltpu.VMEM_SHARED((CNT_PAD,), jnp.float32),
        pltpu.SemaphoreType.DMA,
        pltpu.SemaphoreType.DMA,
        pltpu.SemaphoreType.DMA,
        pltpu.SemaphoreType.DMA,
    ],
)

_sc_agg = pl.kernel(
    _sc_agg_body,
    out_type=jax.ShapeDtypeStruct((NC, N_PAD, H), jnp.float32),
    mesh=_MESH,
    scratch_types=[
        pltpu.VMEM((4, CHUNK), jnp.int32),
        pltpu.VMEM((4, CHUNK), jnp.int32),
        pltpu.VMEM((3, CHUNK, H), jnp.float32),
        pltpu.VMEM_SHARED((N_PAD, H), jnp.float32),
        pltpu.SemaphoreType.DMA,
        pltpu.SemaphoreType.DMA,
        pltpu.SemaphoreType.DMA,
    ],
)


def _layer_body(aggp, h, inv, wl, bl, wr, g, b, out):
    mean = (aggp[0][:N] + aggp[1][:N]) * inv[...]
    z = (jnp.dot(mean, wl[...], preferred_element_type=jnp.float32)
         + jnp.dot(h[...], wr[...], preferred_element_type=jnp.float32)
         + bl[...])
    r = jnp.maximum(z, 0.0)
    m = jnp.mean(r, axis=0, keepdims=True)
    d = r - m
    v = jnp.mean(d * d, axis=0, keepdims=True)
    out[...] = g[...] * d / jnp.sqrt(v + 1e-5) + b[...]


def _layer7_final_body(aggp, h, inv, wl, bl, wr, g, b,
                       batch, ones_n, w1, b1, w2, b2, out):
    mean = (aggp[0][:N] + aggp[1][:N]) * inv[...]
    z = (jnp.dot(mean, wl[...], preferred_element_type=jnp.float32)
         + jnp.dot(h[...], wr[...], preferred_element_type=jnp.float32)
         + bl[...])
    r = jnp.maximum(z, 0.0)
    m = jnp.mean(r, axis=0, keepdims=True)
    d = r - m
    v = jnp.mean(d * d, axis=0, keepdims=True)
    h7 = g[...] * d / jnp.sqrt(v + 1e-5) + b[...]
    onehot = (batch[...] == lax.broadcasted_iota(jnp.int32, (N, G), 1)
              ).astype(jnp.float32)
    dnums = (((0,), (0,)), ((), ()))
    psum = lax.dot_general(onehot, h7, dnums,
                           preferred_element_type=jnp.float32)
    gcnt = lax.dot_general(onehot, ones_n[...], dnums,
                           preferred_element_type=jnp.float32)
    pooled = psum / jnp.maximum(gcnt, 1.0)
    h2 = jnp.maximum(
        jnp.dot(pooled, w1[...], preferred_element_type=jnp.float32)
        + b1[...], 0.0)
    logits = (jnp.dot(h2, w2[...], preferred_element_type=jnp.float32)
              + b2[...])
    mx = jnp.max(logits, axis=-1, keepdims=True)
    s = logits - mx
    lse = jnp.log(jnp.sum(jnp.exp(s), axis=-1, keepdims=True))
    out[...] = s - lse


_tc_layer = pl.pallas_call(
    _layer_body,
    out_shape=jax.ShapeDtypeStruct((N, H), jnp.float32),
)

_tc_layer7_final = pl.pallas_call(
    _layer7_final_body,
    out_shape=jax.ShapeDtypeStruct((G, C), jnp.float32),
)


def kernel(x, edge_index, batch, params):
    src = edge_index[0]
    dst = edge_index[1]
    ones_n = jnp.ones((N, 1), jnp.float32)
    batch2d = batch.reshape(N, 1)

    def w(i):
        p = params['conv%d' % i]
        return (p['Wl'], p['bl'].reshape(1, H), p['Wr'],
                params['bn%d_g' % i].reshape(1, H),
                params['bn%d_b' % i].reshape(1, H))

    aggp, cntp = _sc_agg_cnt(x, src, dst)
    inv = (1.0 / jnp.maximum(cntp[:N] + cntp[CNT_PAD:CNT_PAD + N],
                             1.0)).reshape(N, 1)
    h = x
    for i in range(1, 7):
        if i > 1:
            aggp = _sc_agg(h, src, dst)
        wl, bl, wr, g, b = w(i)
        h = _tc_layer(aggp, h, inv, wl, bl, wr, g, b)

    aggp = _sc_agg(h, src, dst)
    wl, bl, wr, g, b = w(7)
    return _tc_layer7_final(aggp, h, inv, wl, bl, wr, g, b,
                            batch2d, ones_n,
                            params['fc1_W'], params['fc1_b'].reshape(1, H),
                            params['fc2_W'], params['fc2_b'].reshape(1, C))
